# baseline (speedup 1.0000x reference)
import numpy as np
import jax
import jax.numpy as jnp
from jax.experimental import pallas as pl

B, T_IN, D_IN = 8, 512, 80
D = 256
H = 4
DK = D // H
FF = 2048
E = 4
EXP = 1024
KER = 15
T1, F1 = 255, 39
T2, F2 = 127, 19
N_TOK = B * T2
NPAD = 1024
CAP = int(1.25 * N_TOK / E)

_pallas_call = pl.pallas_call


def _ln_in(x, g, b):
    m = jnp.mean(x, axis=-1, keepdims=True)
    d = x - m
    v = jnp.mean(d * d, axis=-1, keepdims=True)
    return d * jax.lax.rsqrt(v + 1e-5) * g + b


def _mm(x, w, b=None, *, act=None, ln=None, residual=None, ys=None, bm=512, bn=None):
    M, K = x.shape
    N = w.shape[1]
    if bn is None:
        if N % 512 == 0:
            bn = 512
        elif N % 256 == 0:
            bn = 256
        else:
            bn = N
    Mp = -(-M // bm) * bm
    if Mp != M:
        x = jnp.pad(x, ((0, Mp - M), (0, 0)))
        if residual is not None:
            residual = jnp.pad(residual, ((0, Mp - M), (0, 0)))
    grid = (Mp // bm, N // bn)
    ins = [x, w]
    specs = [pl.BlockSpec((bm, K), lambda i, j: (i, 0)),
             pl.BlockSpec((K, bn), lambda i, j: (0, j))]
    if b is not None:
        ins.append(b.reshape(1, N))
        specs.append(pl.BlockSpec((1, bn), lambda i, j: (0, j)))
    if ln is not None:
        ins += [ln[0].reshape(1, K), ln[1].reshape(1, K)]
        specs += [pl.BlockSpec((1, K), lambda i, j: (0, 0)),
                  pl.BlockSpec((1, K), lambda i, j: (0, 0))]
    if residual is not None:
        ins.append(residual)
        specs.append(pl.BlockSpec((bm, bn), lambda i, j: (i, j)))

    def body(*refs):
        it = iter(refs[:-1])
        x_ref = next(it)
        w_ref = next(it)
        b_ref = next(it) if b is not None else None
        g_ref = bl_ref = None
        if ln is not None:
            g_ref = next(it)
            bl_ref = next(it)
        r_ref = next(it) if residual is not None else None
        o_ref = refs[-1]
        xb = x_ref[...]
        if ln is not None:
            xb = _ln_in(xb, g_ref[...], bl_ref[...])
        acc = jnp.dot(xb, w_ref[...], preferred_element_type=jnp.float32)
        if b_ref is not None:
            acc = acc + b_ref[...]
        if act == 'relu':
            acc = jnp.maximum(acc, 0.0)
        elif act == 'swish':
            acc = acc * jax.nn.sigmoid(acc)
        if ys is not None:
            acc = acc * ys
        if r_ref is not None:
            acc = r_ref[...] + acc
        o_ref[...] = acc

    out = _pallas_call(
        body,
        grid=grid,
        in_specs=specs,
        out_specs=pl.BlockSpec((bm, bn), lambda i, j: (i, j)),
        out_shape=jax.ShapeDtypeStruct((Mp, N), jnp.float32),
    )(*ins)
    return out[:M] if Mp != M else out


def _attention(qkv, p, u, v, mrow):
    scale = 1.0 / float(np.sqrt(DK))

    def body(qkv_ref, p_ref, u_ref, v_ref, m_ref, o_ref):
        mk = m_ref[0]
        pe = p_ref[...]
        for h in range(H):
            q = qkv_ref[0, :, h * DK:(h + 1) * DK]
            k = qkv_ref[0, :, D + h * DK:D + (h + 1) * DK]
            vv = qkv_ref[0, :, 2 * D + h * DK:2 * D + (h + 1) * DK]
            ph = pe[:, h * DK:(h + 1) * DK]
            uh = u_ref[h:h + 1, :]
            vh = v_ref[h:h + 1, :]
            ac = jax.lax.dot_general(q + uh, k, (((1,), (1,)), ((), ())),
                                     preferred_element_type=jnp.float32)
            bd = jax.lax.dot_general(q + vh, ph, (((1,), (1,)), ((), ())),
                                     preferred_element_type=jnp.float32)
            s = (ac + bd) * scale
            s = jnp.where(mk > 0, s, -1e9)
            s = s - jnp.max(s, axis=-1, keepdims=True)
            es = jnp.exp(s)
            a = es / jnp.sum(es, axis=-1, keepdims=True)
            a = jnp.where(mk > 0, a, 0.0)
            o_ref[0, :, h * DK:(h + 1) * DK] = jnp.dot(
                a, vv, preferred_element_type=jnp.float32)

    return _pallas_call(
        body,
        grid=(B,),
        in_specs=[
            pl.BlockSpec((1, T2, 3 * D), lambda i: (i, 0, 0)),
            pl.BlockSpec((T2, D), lambda i: (0, 0)),
            pl.BlockSpec((H, DK), lambda i: (0, 0)),
            pl.BlockSpec((H, DK), lambda i: (0, 0)),
            pl.BlockSpec((1, 1, T2), lambda i: (i, 0, 0)),
        ],
        out_specs=pl.BlockSpec((1, T2, D), lambda i: (i, 0, 0)),
        out_shape=jax.ShapeDtypeStruct((B, T2, D), jnp.float32),
    )(qkv, p, u, v, mrow)


def _conv_module(x, mcol, g0, b0, pw1, pb1, dw, db, g1, b1, pw2, pb2):
    pad = (KER - 1) // 2

    def body(x_ref, m_ref, g0_ref, b0_ref, pw1_ref, pb1_ref, dw_ref, db_ref,
             g1_ref, b1_ref, pw2_ref, pb2_ref, o_ref):
        xb = x_ref[0]
        mc = m_ref[0]
        y = _ln_in(xb, g0_ref[...], b0_ref[...])
        y = y * mc
        t = jnp.dot(y, pw1_ref[...], preferred_element_type=jnp.float32) + pb1_ref[...]
        y = t[:, :D] * jax.nn.sigmoid(t[:, D:])
        z = jnp.zeros((pad, D), jnp.float32)
        yp = jnp.concatenate([z, y, z], axis=0)
        acc = jnp.zeros((T2, D), jnp.float32)
        for kk in range(KER):
            acc = acc + yp[kk:kk + T2, :] * dw_ref[kk:kk + 1, :]
        y = acc + db_ref[...]
        y = _ln_in(y, g1_ref[...], b1_ref[...])
        y = y * jax.nn.sigmoid(y)
        y = jnp.dot(y, pw2_ref[...], preferred_element_type=jnp.float32) + pb2_ref[...]
        y = y * mc
        o_ref[0] = xb + y

    def full(s):
        return pl.BlockSpec(s, lambda i, _n=len(s): (0,) * _n)

    return _pallas_call(
        body,
        grid=(B,),
        in_specs=[
            pl.BlockSpec((1, T2, D), lambda i: (i, 0, 0)),
            pl.BlockSpec((1, T2, 1), lambda i: (i, 0, 0)),
            full((1, D)), full((1, D)),
            full((D, 2 * D)), full((1, 2 * D)),
            full((KER, D)), full((1, D)),
            full((1, D)), full((1, D)),
            full((D, D)), full((1, D)),
        ],
        out_specs=pl.BlockSpec((1, T2, D), lambda i: (i, 0, 0)),
        out_shape=jax.ShapeDtypeStruct((B, T2, D), jnp.float32),
    )(x, mcol, g0, b0, pw1, pb1, dw, db, g1, b1, pw2, pb2)


def _moe(xin, gg, gb, gw, gbias, w1, b1, w2, b2, fg, fb, ag=None, ab=None):
    extra = ag is not None

    def body(*refs):
        (x_ref, gg_ref, gb_ref, gw_ref, gbias_ref, w1_ref, b1_ref, w2_ref,
         b2_ref, fg_ref, fb_ref) = refs[:11]
        o_ref = refs[-1]
        xb = x_ref[...]
        y = _ln_in(xb, gg_ref[...], gb_ref[...])
        logits = jnp.dot(y, gw_ref[...], preferred_element_type=jnp.float32)
        logits = logits + gbias_ref[...]
        mx = jnp.max(logits, -1, keepdims=True)
        ee = jnp.exp(logits - mx)
        probs = ee / jnp.sum(ee, -1, keepdims=True)
        gval = jnp.max(probs, -1, keepdims=True)
        iota_e = jax.lax.broadcasted_iota(jnp.int32, (NPAD, E), 1)
        cand = jnp.where(probs >= gval, iota_e, E)
        fidx = jnp.min(cand, -1, keepdims=True)
        row = jax.lax.broadcasted_iota(jnp.int32, (NPAD, 1), 0)
        validf = (row < N_TOK).astype(jnp.float32)
        onehot = (iota_e == fidx).astype(jnp.float32) * validf
        r2 = jax.lax.broadcasted_iota(jnp.int32, (NPAD, NPAD), 0)
        c2 = jax.lax.broadcasted_iota(jnp.int32, (NPAD, NPAD), 1)
        tri = (c2 <= r2).astype(jnp.float32)
        csum = jnp.dot(tri, onehot, preferred_element_type=jnp.float32)
        mypos = jnp.sum(csum * onehot, -1, keepdims=True) - 1.0
        keep = (mypos < CAP).astype(jnp.float32)
        comb = jnp.zeros((NPAD, D), jnp.float32)
        for ei in range(E):
            h = jnp.dot(y, w1_ref[ei], preferred_element_type=jnp.float32) + b1_ref[ei]
            h = h * jax.nn.sigmoid(h)
            oo = jnp.dot(h, w2_ref[ei], preferred_element_type=jnp.float32) + b2_ref[ei]
            comb = comb + onehot[:, ei:ei + 1] * oo
        out = comb * (gval * keep)
        x4 = xb + 0.5 * out
        x4 = _ln_in(x4, fg_ref[...], fb_ref[...])
        if extra:
            ag_ref, ab_ref = refs[11], refs[12]
            x4 = _ln_in(x4, ag_ref[...], ab_ref[...])
        o_ref[...] = x4

    ins = [xin, gg, gb, gw, gbias, w1, b1, w2, b2, fg, fb]
    if extra:
        ins += [ag, ab]
    return _pallas_call(
        body,
        out_shape=jax.ShapeDtypeStruct((NPAD, D), jnp.float32),
    )(*ins)


def _rel_pos_enc_np(t):
    pos = np.arange(t)[:, None].astype(np.float32)
    div = np.exp(np.arange(0, D, 2).astype(np.float32) * -(np.log(10000.0) / D))
    pe = np.zeros((t, D), np.float32)
    pe[:, 0::2] = np.sin(pos * div)
    pe[:, 1::2] = np.cos(pos * div)
    return pe


def _subsample(xs, params):
    pats = []
    for kh in range(3):
        for kw in range(3):
            pats.append(xs[:, kh::2, :][:, :T1, kw::2][:, :, :F1])
    p1 = jnp.stack(pats, axis=-1).reshape(B * T1 * F1, 9)
    w1 = jnp.transpose(params['conv1_w'], (2, 3, 1, 0)).reshape(9, D)
    y1 = _mm(p1, w1, params['conv1_b'], act='relu', bm=1024)
    y1 = y1.reshape(B, T1, F1, D)

    pats2 = []
    for kh in range(3):
        for kw in range(3):
            pats2.append(y1[:, kh::2, :, :][:, :T2, kw::2, :][:, :, :F2, :])
    p2 = jnp.stack(pats2, axis=3).reshape(B * T2 * F2, 9 * D)
    w2 = jnp.transpose(params['conv2_w'], (2, 3, 1, 0)).reshape(9 * D, D)
    y2 = _mm(p2, w2, params['conv2_b'], act='relu', bm=512)
    y2 = y2.reshape(B, T2, F2 * D)

    out_w = params['out_w'].reshape(D, F2, D).transpose(1, 0, 2).reshape(F2 * D, D)
    x0 = _mm(y2.reshape(B * T2, F2 * D), out_w, params['out_b'],
             ys=float(np.sqrt(D)), bm=512)
    return x0


def _layer(x0, pos_emb, mrow, mcol, L, after=None):
    qkv_w = jnp.concatenate([L['att_q_w'], L['att_k_w'], L['att_v_w']], axis=1)
    qkv_b = jnp.concatenate([L['att_q_b'], L['att_k_b'], L['att_v_b']], axis=0)

    h = _mm(x0, L['mac_w1'], L['mac_b1'], ln=(L['ln_ff_mac_g'], L['ln_ff_mac_b']),
            act='swish')
    x1 = _mm(h, L['mac_w2'], L['mac_b2'], residual=x0, ys=0.5)
    qkv = _mm(x1, qkv_w, qkv_b, ln=(L['ln_mha_g'], L['ln_mha_b']))
    p = _mm(pos_emb, L['att_pos_w'], bm=128)
    att = _attention(qkv.reshape(B, T2, 3 * D), p, L['pos_u'], L['pos_v'], mrow)
    x2 = _mm(att.reshape(N_TOK, D), L['att_o_w'], L['att_o_b'], residual=x1)
    x3 = _conv_module(x2.reshape(B, T2, D), mcol,
                      L['ln_conv_g'].reshape(1, D), L['ln_conv_b'].reshape(1, D),
                      L['conv_pw1_w'], L['conv_pw1_b'].reshape(1, 2 * D),
                      jnp.transpose(L['conv_dw_w'].reshape(D, KER)),
                      L['conv_dw_b'].reshape(1, D),
                      L['conv_ln_g'].reshape(1, D), L['conv_ln_b'].reshape(1, D),
                      L['conv_pw2_w'], L['conv_pw2_b'].reshape(1, D))
    x3f = jnp.pad(x3.reshape(N_TOK, D), ((0, NPAD - N_TOK), (0, 0)))
    kwargs = {}
    if after is not None:
        kwargs = dict(ag=after[0].reshape(1, D), ab=after[1].reshape(1, D))
    x4 = _moe(x3f,
              L['ln_ff_g'].reshape(1, D), L['ln_ff_b'].reshape(1, D),
              L['gate_w'], L['gate_b'].reshape(1, E),
              L['exp_w1'], L['exp_b1'].reshape(E, 1, EXP),
              L['exp_w2'], L['exp_b2'].reshape(E, 1, D),
              L['ln_final_g'].reshape(1, D), L['ln_final_b'].reshape(1, D),
              **kwargs)
    return x4[:N_TOK]


def _forward(xs, xs_lens, params):
    tm = jnp.arange(T_IN)[None, :] < xs_lens[:, None]
    masks = tm[:, 2::2][:, 2::2]
    mrow = masks.astype(jnp.float32)[:, None, :]
    mcol = masks.astype(jnp.float32)[:, :, None]
    pos_emb = jnp.asarray(_rel_pos_enc_np(T2))

    x = _subsample(xs, params)
    layers = params['layers']
    x = _layer(x, pos_emb, mrow, mcol, layers[0])
    x = _layer(x, pos_emb, mrow, mcol, layers[1],
               after=(params['after_g'], params['after_b']))
    return x.reshape(B, T2, D), masks[:, None, :]


kernel = jax.jit(_forward)


# speedup vs baseline: 7.0915x; 7.0915x over previous
import numpy as np
import jax
import jax.numpy as jnp
from jax.experimental import pallas as pl

B, T_IN, D_IN = 8, 512, 80
D = 256
H = 4
DK = D // H
FF = 2048
E = 4
EXP = 1024
KER = 15
T1, F1 = 255, 39
T2, F2 = 127, 19
N_TOK = B * T2
NPAD = 1024
CAP = int(1.25 * N_TOK / E)
CAPP = 320

_pallas_call = pl.pallas_call


def _ln_in(x, g, b):
    m = jnp.mean(x, axis=-1, keepdims=True)
    d = x - m
    v = jnp.mean(d * d, axis=-1, keepdims=True)
    return d * jax.lax.rsqrt(v + 1e-5) * g + b


def _bdot(a, b):
    return jax.lax.dot_general(
        a.astype(jnp.bfloat16), b.astype(jnp.bfloat16),
        (((a.ndim - 1,), (0,)), ((), ())),
        preferred_element_type=jnp.float32)


def _bdot_t(a, b):
    return jax.lax.dot_general(
        a.astype(jnp.bfloat16), b.astype(jnp.bfloat16),
        (((1,), (1,)), ((), ())),
        preferred_element_type=jnp.float32)


def _mm(x, w, b=None, *, act=None, ln=None, residual=None, ys=None, bm=512, bn=None):
    M, K = x.shape
    N = w.shape[1]
    if bn is None:
        if N % 512 == 0:
            bn = 512
        elif N % 256 == 0:
            bn = 256
        else:
            bn = N
    Mp = -(-M // bm) * bm
    if Mp != M:
        x = jnp.pad(x, ((0, Mp - M), (0, 0)))
        if residual is not None:
            residual = jnp.pad(residual, ((0, Mp - M), (0, 0)))
    grid = (Mp // bm, N // bn)
    ins = [x, w]
    specs = [pl.BlockSpec((bm, K), lambda i, j: (i, 0)),
             pl.BlockSpec((K, bn), lambda i, j: (0, j))]
    if b is not None:
        ins.append(b.reshape(1, N))
        specs.append(pl.BlockSpec((1, bn), lambda i, j: (0, j)))
    if ln is not None:
        ins += [ln[0].reshape(1, K), ln[1].reshape(1, K)]
        specs += [pl.BlockSpec((1, K), lambda i, j: (0, 0)),
                  pl.BlockSpec((1, K), lambda i, j: (0, 0))]
    if residual is not None:
        ins.append(residual)
        specs.append(pl.BlockSpec((bm, bn), lambda i, j: (i, j)))

    def body(*refs):
        it = iter(refs[:-1])
        x_ref = next(it)
        w_ref = next(it)
        b_ref = next(it) if b is not None else None
        g_ref = bl_ref = None
        if ln is not None:
            g_ref = next(it)
            bl_ref = next(it)
        r_ref = next(it) if residual is not None else None
        o_ref = refs[-1]
        xb = x_ref[...]
        if ln is not None:
            xb = _ln_in(xb, g_ref[...], bl_ref[...])
        acc = _bdot(xb, w_ref[...])
        if b_ref is not None:
            acc = acc + b_ref[...]
        if act == 'relu':
            acc = jnp.maximum(acc, 0.0)
        elif act == 'swish':
            acc = acc * jax.nn.sigmoid(acc)
        if ys is not None:
            acc = acc * ys
        if r_ref is not None:
            acc = r_ref[...] + acc
        o_ref[...] = acc

    out = _pallas_call(
        body,
        grid=grid,
        in_specs=specs,
        out_specs=pl.BlockSpec((bm, bn), lambda i, j: (i, j)),
        out_shape=jax.ShapeDtypeStruct((Mp, N), jnp.float32),
    )(*ins)
    return out[:M] if Mp != M else out


def _attention(qkv, p, u, v, mrow):
    scale = 1.0 / float(np.sqrt(DK))

    def body(qkv_ref, p_ref, u_ref, v_ref, m_ref, o_ref):
        mk = m_ref[0]
        pe = p_ref[...]
        for h in range(H):
            q = qkv_ref[0, :, h * DK:(h + 1) * DK]
            k = qkv_ref[0, :, D + h * DK:D + (h + 1) * DK]
            vv = qkv_ref[0, :, 2 * D + h * DK:2 * D + (h + 1) * DK]
            ph = pe[:, h * DK:(h + 1) * DK]
            uh = u_ref[h:h + 1, :]
            vh = v_ref[h:h + 1, :]
            ac = _bdot_t(q + uh, k)
            bd = _bdot_t(q + vh, ph)
            s = (ac + bd) * scale
            s = jnp.where(mk > 0, s, -1e9)
            s = s - jnp.max(s, axis=-1, keepdims=True)
            es = jnp.exp(s)
            a = es / jnp.sum(es, axis=-1, keepdims=True)
            a = jnp.where(mk > 0, a, 0.0)
            o_ref[0, :, h * DK:(h + 1) * DK] = _bdot(a, vv)

    return _pallas_call(
        body,
        grid=(B,),
        in_specs=[
            pl.BlockSpec((1, T2, 3 * D), lambda i: (i, 0, 0)),
            pl.BlockSpec((T2, D), lambda i: (0, 0)),
            pl.BlockSpec((H, DK), lambda i: (0, 0)),
            pl.BlockSpec((H, DK), lambda i: (0, 0)),
            pl.BlockSpec((1, 1, T2), lambda i: (i, 0, 0)),
        ],
        out_specs=pl.BlockSpec((1, T2, D), lambda i: (i, 0, 0)),
        out_shape=jax.ShapeDtypeStruct((B, T2, D), jnp.float32),
    )(qkv, p, u, v, mrow)


def _conv_module(x, mcol, g0, b0, pw1, pb1, dw, db, g1, b1, pw2, pb2):
    pad = (KER - 1) // 2

    def body(x_ref, m_ref, g0_ref, b0_ref, pw1_ref, pb1_ref, dw_ref, db_ref,
             g1_ref, b1_ref, pw2_ref, pb2_ref, o_ref):
        xb = x_ref[0]
        mc = m_ref[0]
        y = _ln_in(xb, g0_ref[...], b0_ref[...])
        y = y * mc
        t = _bdot(y, pw1_ref[...]) + pb1_ref[...]
        y = t[:, :D] * jax.nn.sigmoid(t[:, D:])
        z = jnp.zeros((pad, D), jnp.float32)
        yp = jnp.concatenate([z, y, z], axis=0)
        acc = jnp.zeros((T2, D), jnp.float32)
        for kk in range(KER):
            acc = acc + yp[kk:kk + T2, :] * dw_ref[kk:kk + 1, :]
        y = acc + db_ref[...]
        y = _ln_in(y, g1_ref[...], b1_ref[...])
        y = y * jax.nn.sigmoid(y)
        y = _bdot(y, pw2_ref[...]) + pb2_ref[...]
        y = y * mc
        o_ref[0] = xb + y

    def full(s):
        return pl.BlockSpec(s, lambda i, _n=len(s): (0,) * _n)

    return _pallas_call(
        body,
        grid=(B,),
        in_specs=[
            pl.BlockSpec((1, T2, D), lambda i: (i, 0, 0)),
            pl.BlockSpec((1, T2, 1), lambda i: (i, 0, 0)),
            full((1, D)), full((1, D)),
            full((D, 2 * D)), full((1, 2 * D)),
            full((KER, D)), full((1, D)),
            full((1, D)), full((1, D)),
            full((D, D)), full((1, D)),
        ],
        out_specs=pl.BlockSpec((1, T2, D), lambda i: (i, 0, 0)),
        out_shape=jax.ShapeDtypeStruct((B, T2, D), jnp.float32),
    )(x, mcol, g0, b0, pw1, pb1, dw, db, g1, b1, pw2, pb2)


def _moe(xin, gg, gb, gw, gbias, w1, b1, w2, b2, fg, fb, ag=None, ab=None):
    extra = ag is not None

    def body(*refs):
        (x_ref, gg_ref, gb_ref, gw_ref, gbias_ref, w1_ref, b1_ref, w2_ref,
         b2_ref, fg_ref, fb_ref) = refs[:11]
        o_ref = refs[-1]
        xb = x_ref[...]
        y = _ln_in(xb, gg_ref[...], gb_ref[...])
        logits = jnp.dot(y, gw_ref[...], preferred_element_type=jnp.float32)
        logits = logits + gbias_ref[...]
        mx = jnp.max(logits, -1, keepdims=True)
        ee = jnp.exp(logits - mx)
        probs = ee / jnp.sum(ee, -1, keepdims=True)
        gval = jnp.max(probs, -1, keepdims=True)
        iota_e = jax.lax.broadcasted_iota(jnp.int32, (NPAD, E), 1)
        cand = jnp.where(probs >= gval, iota_e, E)
        fidx = jnp.min(cand, -1, keepdims=True)
        row = jax.lax.broadcasted_iota(jnp.int32, (NPAD, 1), 0)
        validf = (row < N_TOK).astype(jnp.float32)
        onehot = (iota_e == fidx).astype(jnp.float32) * validf
        r2 = jax.lax.broadcasted_iota(jnp.int32, (NPAD, NPAD), 0)
        c2 = jax.lax.broadcasted_iota(jnp.int32, (NPAD, NPAD), 1)
        tri = (c2 <= r2).astype(jnp.float32)
        csum = jnp.dot(tri, onehot, preferred_element_type=jnp.float32)
        mypos = jnp.sum(csum * onehot, -1, keepdims=True) - 1.0
        keep = (mypos < CAP).astype(jnp.float32)
        kept = keep * validf * (mypos >= 0.0).astype(jnp.float32)
        dst = fidx.astype(jnp.float32) * CAPP + mypos
        dst = jnp.where(kept > 0, dst, -1.0)
        slot_iota = jax.lax.broadcasted_iota(jnp.int32, (NPAD, E * CAPP), 1)
        Dt = (slot_iota.astype(jnp.float32) == dst).astype(jnp.float32)
        xslots = jax.lax.dot_general(
            Dt.astype(jnp.bfloat16), y.astype(jnp.bfloat16),
            (((0,), (0,)), ((), ())), preferred_element_type=jnp.float32)
        oparts = []
        for ei in range(E):
            ye = xslots[ei * CAPP:(ei + 1) * CAPP]
            h = _bdot(ye, w1_ref[ei]) + b1_ref[ei]
            h = h * jax.nn.sigmoid(h)
            oparts.append(_bdot(h, w2_ref[ei]) + b2_ref[ei])
        oslots = jnp.concatenate(oparts, axis=0)
        comb = _bdot(Dt, oslots)
        out = comb * gval
        x4 = xb + 0.5 * out
        x4 = _ln_in(x4, fg_ref[...], fb_ref[...])
        if extra:
            ag_ref, ab_ref = refs[11], refs[12]
            x4 = _ln_in(x4, ag_ref[...], ab_ref[...])
        o_ref[...] = x4

    ins = [xin, gg, gb, gw, gbias, w1, b1, w2, b2, fg, fb]
    if extra:
        ins += [ag, ab]
    return _pallas_call(
        body,
        out_shape=jax.ShapeDtypeStruct((NPAD, D), jnp.float32),
    )(*ins)


def _rel_pos_enc_np(t):
    pos = np.arange(t)[:, None].astype(np.float32)
    div = np.exp(np.arange(0, D, 2).astype(np.float32) * -(np.log(10000.0) / D))
    pe = np.zeros((t, D), np.float32)
    pe[:, 0::2] = np.sin(pos * div)
    pe[:, 1::2] = np.cos(pos * div)
    return pe


def _subsample(xs, params):
    xsp = jnp.pad(xs, ((0, 0), (0, 4), (0, 4)))
    phase_stacks = []
    for p in range(2):
        for q in range(2):
            pats = []
            for kh in range(3):
                for kw in range(3):
                    sl = xsp[:, 2 * p + kh::4, :][:, :128, 2 * q + kw::4][:, :, :20]
                    pats.append(sl)
            phase_stacks.append(jnp.stack(pats, axis=-1))
    patches = jnp.concatenate(phase_stacks, axis=1)
    w1r = jnp.transpose(params['conv1_w'], (2, 3, 1, 0)).reshape(9, D)
    w2r = jnp.transpose(params['conv2_w'], (2, 3, 1, 0)).reshape(9, D, D)
    owr = params['out_w'].reshape(D, F2, D).transpose(1, 0, 2)
    b1 = params['conv1_b'].reshape(1, 1, D)
    b2 = params['conv2_b'].reshape(1, 1, D)
    ob = params['out_b'].reshape(1, D)
    scale = float(np.sqrt(D))

    def body(p_ref, w1_ref, b1_ref, w2_ref, b2_ref, ow_ref, ob_ref, o_ref):
        ph = _bdot(p_ref[0], w1_ref[...]) + b1_ref[...]
        ph = jnp.maximum(ph, 0.0)
        acc = jnp.zeros((T2, F2, D), jnp.float32)
        for kh in range(3):
            p = kh & 1
            to = kh >> 1
            for kw in range(3):
                q = kw & 1
                fo = kw >> 1
                base = (2 * p + q) * 128 + to
                sl = ph[base:base + T2, fo:fo + F2, :]
                acc = acc + _bdot(sl, w2_ref[kh * 3 + kw])
        z = jnp.maximum(acc + b2_ref[...], 0.0)
        out = ob_ref[...]
        for f in range(F2):
            out = out + _bdot(z[:, f, :], ow_ref[f])
        o_ref[0] = out * scale

    x0 = _pallas_call(
        body,
        grid=(B,),
        in_specs=[
            pl.BlockSpec((1, 512, 20, 9), lambda i: (i, 0, 0, 0)),
            pl.BlockSpec((9, D), lambda i: (0, 0)),
            pl.BlockSpec((1, 1, D), lambda i: (0, 0, 0)),
            pl.BlockSpec((9, D, D), lambda i: (0, 0, 0)),
            pl.BlockSpec((1, 1, D), lambda i: (0, 0, 0)),
            pl.BlockSpec((F2, D, D), lambda i: (0, 0, 0)),
            pl.BlockSpec((1, D), lambda i: (0, 0)),
        ],
        out_specs=pl.BlockSpec((1, T2, D), lambda i: (i, 0, 0)),
        out_shape=jax.ShapeDtypeStruct((B, T2, D), jnp.float32),
    )(patches, w1r, b1, w2r, b2, owr, ob)
    return x0.reshape(N_TOK, D)


def _layer(x0, pos_emb, mrow, mcol, L, after=None):
    qkv_w = jnp.concatenate([L['att_q_w'], L['att_k_w'], L['att_v_w']], axis=1)
    qkv_b = jnp.concatenate([L['att_q_b'], L['att_k_b'], L['att_v_b']], axis=0)

    h = _mm(x0, L['mac_w1'], L['mac_b1'], ln=(L['ln_ff_mac_g'], L['ln_ff_mac_b']),
            act='swish')
    x1 = _mm(h, L['mac_w2'], L['mac_b2'], residual=x0, ys=0.5)
    qkv = _mm(x1, qkv_w, qkv_b, ln=(L['ln_mha_g'], L['ln_mha_b']))
    p = _mm(pos_emb, L['att_pos_w'], bm=128)
    att = _attention(qkv.reshape(B, T2, 3 * D), p, L['pos_u'], L['pos_v'], mrow)
    x2 = _mm(att.reshape(N_TOK, D), L['att_o_w'], L['att_o_b'], residual=x1)
    x3 = _conv_module(x2.reshape(B, T2, D), mcol,
                      L['ln_conv_g'].reshape(1, D), L['ln_conv_b'].reshape(1, D),
                      L['conv_pw1_w'], L['conv_pw1_b'].reshape(1, 2 * D),
                      jnp.transpose(L['conv_dw_w'].reshape(D, KER)),
                      L['conv_dw_b'].reshape(1, D),
                      L['conv_ln_g'].reshape(1, D), L['conv_ln_b'].reshape(1, D),
                      L['conv_pw2_w'], L['conv_pw2_b'].reshape(1, D))
    x3f = jnp.pad(x3.reshape(N_TOK, D), ((0, NPAD - N_TOK), (0, 0)))
    kwargs = {}
    if after is not None:
        kwargs = dict(ag=after[0].reshape(1, D), ab=after[1].reshape(1, D))
    x4 = _moe(x3f,
              L['ln_ff_g'].reshape(1, D), L['ln_ff_b'].reshape(1, D),
              L['gate_w'], L['gate_b'].reshape(1, E),
              L['exp_w1'], L['exp_b1'].reshape(E, 1, EXP),
              L['exp_w2'], L['exp_b2'].reshape(E, 1, D),
              L['ln_final_g'].reshape(1, D), L['ln_final_b'].reshape(1, D),
              **kwargs)
    return x4[:N_TOK]


def _forward(xs, xs_lens, params):
    tm = jnp.arange(T_IN)[None, :] < xs_lens[:, None]
    masks = tm[:, 2::2][:, 2::2]
    mrow = masks.astype(jnp.float32)[:, None, :]
    mcol = masks.astype(jnp.float32)[:, :, None]
    pos_emb = jnp.asarray(_rel_pos_enc_np(T2))

    x = _subsample(xs, params)
    layers = params['layers']
    x = _layer(x, pos_emb, mrow, mcol, layers[0])
    x = _layer(x, pos_emb, mrow, mcol, layers[1],
               after=(params['after_g'], params['after_b']))
    return x.reshape(B, T2, D), masks[:, None, :]


kernel = jax.jit(_forward)


# speedup vs baseline: 8.9730x; 1.2653x over previous
import functools

import numpy as np
import jax
import jax.numpy as jnp
from jax.experimental import pallas as pl
from jax.experimental.pallas import tpu as pltpu
from jax.experimental.pallas import tpu_sc as plsc

B, T_IN, D_IN = 8, 512, 80
D = 256
H = 4
DK = D // H
FF = 2048
E = 4
EXP = 1024
KER = 15
T1, F1 = 255, 39
T2, F2 = 127, 19
N_TOK = B * T2
NPAD = 1024
CAP = int(1.25 * N_TOK / E)
CAPP = 320

_pallas_call = pl.pallas_call


def _ln_in(x, g, b):
    m = jnp.mean(x, axis=-1, keepdims=True)
    d = x - m
    v = jnp.mean(d * d, axis=-1, keepdims=True)
    return d * jax.lax.rsqrt(v + 1e-5) * g + b


def _bdot(a, b):
    return jax.lax.dot_general(
        a.astype(jnp.bfloat16), b.astype(jnp.bfloat16),
        (((a.ndim - 1,), (0,)), ((), ())),
        preferred_element_type=jnp.float32)


def _bdot_t(a, b):
    return jax.lax.dot_general(
        a.astype(jnp.bfloat16), b.astype(jnp.bfloat16),
        (((1,), (1,)), ((), ())),
        preferred_element_type=jnp.float32)


def _mm(x, w, b=None, *, act=None, ln=None, residual=None, ys=None, bm=512, bn=None):
    M, K = x.shape
    N = w.shape[1]
    if bn is None:
        if N % 512 == 0:
            bn = 512
        elif N % 256 == 0:
            bn = 256
        else:
            bn = N
    Mp = -(-M // bm) * bm
    if Mp != M:
        x = jnp.pad(x, ((0, Mp - M), (0, 0)))
        if residual is not None:
            residual = jnp.pad(residual, ((0, Mp - M), (0, 0)))
    grid = (Mp // bm, N // bn)
    ins = [x, w]
    specs = [pl.BlockSpec((bm, K), lambda i, j: (i, 0)),
             pl.BlockSpec((K, bn), lambda i, j: (0, j))]
    if b is not None:
        ins.append(b.reshape(1, N))
        specs.append(pl.BlockSpec((1, bn), lambda i, j: (0, j)))
    if ln is not None:
        ins += [ln[0].reshape(1, K), ln[1].reshape(1, K)]
        specs += [pl.BlockSpec((1, K), lambda i, j: (0, 0)),
                  pl.BlockSpec((1, K), lambda i, j: (0, 0))]
    if residual is not None:
        ins.append(residual)
        specs.append(pl.BlockSpec((bm, bn), lambda i, j: (i, j)))

    def body(*refs):
        it = iter(refs[:-1])
        x_ref = next(it)
        w_ref = next(it)
        b_ref = next(it) if b is not None else None
        g_ref = bl_ref = None
        if ln is not None:
            g_ref = next(it)
            bl_ref = next(it)
        r_ref = next(it) if residual is not None else None
        o_ref = refs[-1]
        xb = x_ref[...]
        if ln is not None:
            xb = _ln_in(xb, g_ref[...], bl_ref[...])
        acc = _bdot(xb, w_ref[...])
        if b_ref is not None:
            acc = acc + b_ref[...]
        if act == 'relu':
            acc = jnp.maximum(acc, 0.0)
        elif act == 'swish':
            acc = acc * jax.nn.sigmoid(acc)
        if ys is not None:
            acc = acc * ys
        if r_ref is not None:
            acc = r_ref[...] + acc
        o_ref[...] = acc

    out = _pallas_call(
        body,
        grid=grid,
        in_specs=specs,
        out_specs=pl.BlockSpec((bm, bn), lambda i, j: (i, j)),
        out_shape=jax.ShapeDtypeStruct((Mp, N), jnp.float32),
    )(*ins)
    return out[:M] if Mp != M else out


def _attention(qkv, p, u, v, mrow):
    scale = 1.0 / float(np.sqrt(DK))

    def body(qkv_ref, p_ref, u_ref, v_ref, m_ref, o_ref):
        mk = m_ref[0]
        pe = p_ref[...]
        for h in range(H):
            q = qkv_ref[0, :, h * DK:(h + 1) * DK]
            k = qkv_ref[0, :, D + h * DK:D + (h + 1) * DK]
            vv = qkv_ref[0, :, 2 * D + h * DK:2 * D + (h + 1) * DK]
            ph = pe[:, h * DK:(h + 1) * DK]
            uh = u_ref[h:h + 1, :]
            vh = v_ref[h:h + 1, :]
            ac = _bdot_t(q + uh, k)
            bd = _bdot_t(q + vh, ph)
            s = (ac + bd) * scale
            s = jnp.where(mk > 0, s, -1e9)
            s = s - jnp.max(s, axis=-1, keepdims=True)
            es = jnp.exp(s)
            a = es / jnp.sum(es, axis=-1, keepdims=True)
            a = jnp.where(mk > 0, a, 0.0)
            o_ref[0, :, h * DK:(h + 1) * DK] = _bdot(a, vv)

    return _pallas_call(
        body,
        grid=(B,),
        in_specs=[
            pl.BlockSpec((1, T2, 3 * D), lambda i: (i, 0, 0)),
            pl.BlockSpec((T2, D), lambda i: (0, 0)),
            pl.BlockSpec((H, DK), lambda i: (0, 0)),
            pl.BlockSpec((H, DK), lambda i: (0, 0)),
            pl.BlockSpec((1, 1, T2), lambda i: (i, 0, 0)),
        ],
        out_specs=pl.BlockSpec((1, T2, D), lambda i: (i, 0, 0)),
        out_shape=jax.ShapeDtypeStruct((B, T2, D), jnp.float32),
    )(qkv, p, u, v, mrow)


def _conv_module(x, mcol, g0, b0, pw1, pb1, dw, db, g1, b1, pw2, pb2):
    pad = (KER - 1) // 2

    def body(x_ref, m_ref, g0_ref, b0_ref, pw1_ref, pb1_ref, dw_ref, db_ref,
             g1_ref, b1_ref, pw2_ref, pb2_ref, o_ref):
        xb = x_ref[0]
        mc = m_ref[0]
        y = _ln_in(xb, g0_ref[...], b0_ref[...])
        y = y * mc
        t = _bdot(y, pw1_ref[...]) + pb1_ref[...]
        y = t[:, :D] * jax.nn.sigmoid(t[:, D:])
        z = jnp.zeros((pad, D), jnp.float32)
        yp = jnp.concatenate([z, y, z], axis=0)
        acc = jnp.zeros((T2, D), jnp.float32)
        for kk in range(KER):
            acc = acc + yp[kk:kk + T2, :] * dw_ref[kk:kk + 1, :]
        y = acc + db_ref[...]
        y = _ln_in(y, g1_ref[...], b1_ref[...])
        y = y * jax.nn.sigmoid(y)
        y = _bdot(y, pw2_ref[...]) + pb2_ref[...]
        y = y * mc
        o_ref[0] = xb + y

    def full(s):
        return pl.BlockSpec(s, lambda i, _n=len(s): (0,) * _n)

    return _pallas_call(
        body,
        grid=(B,),
        in_specs=[
            pl.BlockSpec((1, T2, D), lambda i: (i, 0, 0)),
            pl.BlockSpec((1, T2, 1), lambda i: (i, 0, 0)),
            full((1, D)), full((1, D)),
            full((D, 2 * D)), full((1, 2 * D)),
            full((KER, D)), full((1, D)),
            full((1, D)), full((1, D)),
            full((D, D)), full((1, D)),
        ],
        out_specs=pl.BlockSpec((1, T2, D), lambda i: (i, 0, 0)),
        out_shape=jax.ShapeDtypeStruct((B, T2, D), jnp.float32),
    )(x, mcol, g0, b0, pw1, pb1, dw, db, g1, b1, pw2, pb2)


def _moe(xin, gg, gb, gw, gbias, w1, b1, w2, b2, fg, fb, ag=None, ab=None):
    extra = ag is not None

    def body(*refs):
        (x_ref, gg_ref, gb_ref, gw_ref, gbias_ref, w1_ref, b1_ref, w2_ref,
         b2_ref, fg_ref, fb_ref) = refs[:11]
        o_ref = refs[-1]
        xb = x_ref[...]
        y = _ln_in(xb, gg_ref[...], gb_ref[...])
        logits = jnp.dot(y, gw_ref[...], preferred_element_type=jnp.float32)
        logits = logits + gbias_ref[...]
        mx = jnp.max(logits, -1, keepdims=True)
        ee = jnp.exp(logits - mx)
        probs = ee / jnp.sum(ee, -1, keepdims=True)
        gval = jnp.max(probs, -1, keepdims=True)
        iota_e = jax.lax.broadcasted_iota(jnp.int32, (NPAD, E), 1)
        cand = jnp.where(probs >= gval, iota_e, E)
        fidx = jnp.min(cand, -1, keepdims=True)
        row = jax.lax.broadcasted_iota(jnp.int32, (NPAD, 1), 0)
        validf = (row < N_TOK).astype(jnp.float32)
        onehot = (iota_e == fidx).astype(jnp.float32) * validf
        r2 = jax.lax.broadcasted_iota(jnp.int32, (NPAD, NPAD), 0)
        c2 = jax.lax.broadcasted_iota(jnp.int32, (NPAD, NPAD), 1)
        tri = (c2 <= r2).astype(jnp.float32)
        csum = jnp.dot(tri, onehot, preferred_element_type=jnp.float32)
        mypos = jnp.sum(csum * onehot, -1, keepdims=True) - 1.0
        keep = (mypos < CAP).astype(jnp.float32)
        kept = keep * validf * (mypos >= 0.0).astype(jnp.float32)
        dst = fidx.astype(jnp.float32) * CAPP + mypos
        dst = jnp.where(kept > 0, dst, -1.0)
        slot_iota = jax.lax.broadcasted_iota(jnp.int32, (NPAD, E * CAPP), 1)
        Dt = (slot_iota.astype(jnp.float32) == dst).astype(jnp.float32)
        xslots = jax.lax.dot_general(
            Dt.astype(jnp.bfloat16), y.astype(jnp.bfloat16),
            (((0,), (0,)), ((), ())), preferred_element_type=jnp.float32)
        oparts = []
        for ei in range(E):
            ye = xslots[ei * CAPP:(ei + 1) * CAPP]
            h = _bdot(ye, w1_ref[ei]) + b1_ref[ei]
            h = h * jax.nn.sigmoid(h)
            oparts.append(_bdot(h, w2_ref[ei]) + b2_ref[ei])
        oslots = jnp.concatenate(oparts, axis=0)
        comb = _bdot(Dt, oslots)
        out = comb * gval
        x4 = xb + 0.5 * out
        x4 = _ln_in(x4, fg_ref[...], fb_ref[...])
        if extra:
            ag_ref, ab_ref = refs[11], refs[12]
            x4 = _ln_in(x4, ag_ref[...], ab_ref[...])
        o_ref[...] = x4

    ins = [xin, gg, gb, gw, gbias, w1, b1, w2, b2, fg, fb]
    if extra:
        ins += [ag, ab]
    return _pallas_call(
        body,
        out_shape=jax.ShapeDtypeStruct((NPAD, D), jnp.float32),
    )(*ins)


def _layer_fused(x, mrow, mcol, pemb, Lw, after=None):
    extra = after is not None
    pad = (KER - 1) // 2
    iscale = 1.0 / float(np.sqrt(DK))

    names = ['mac_w1', 'mac_b1', 'mac_w2', 'mac_b2', 'qkv_w', 'qkv_b',
             'att_pos_w', 'pos_u', 'pos_v', 'att_o_w', 'att_o_b',
             'conv_pw1_w', 'conv_pw1_b', 'conv_dw', 'conv_dw_b',
             'conv_ln_g', 'conv_ln_b', 'conv_pw2_w', 'conv_pw2_b',
             'gate_w', 'gate_b', 'exp_w1', 'exp_b1', 'exp_w2', 'exp_b2',
             'ln_ff_mac_g', 'ln_ff_mac_b', 'ln_mha_g', 'ln_mha_b',
             'ln_conv_g', 'ln_conv_b', 'ln_ff_g', 'ln_ff_b',
             'ln_final_g', 'ln_final_b']
    if extra:
        names += ['after_g', 'after_b']

    def body(*refs):
        x_ref, mrow_ref, mcol_ref, pemb_ref = refs[:4]
        W = dict(zip(names, refs[4:4 + len(names)]))
        o_ref = refs[4 + len(names)]
        scr = refs[5 + len(names)]

        xb = x_ref[...]
        mc = mcol_ref[...]
        scr[...] = jnp.zeros((NPAD, D), jnp.float32)

        y = _ln_in(xb, W['ln_ff_mac_g'][...], W['ln_ff_mac_b'][...])
        h = _bdot(y, W['mac_w1'][...]) + W['mac_b1'][...]
        h = h * jax.nn.sigmoid(h)
        x1 = xb + 0.5 * (_bdot(h, W['mac_w2'][...]) + W['mac_b2'][...])

        y = _ln_in(x1, W['ln_mha_g'][...], W['ln_mha_b'][...])
        qkv = _bdot(y, W['qkv_w'][...]) + W['qkv_b'][...]
        pp = _bdot(pemb_ref[...], W['att_pos_w'][...])
        for b8 in range(B):
            r0 = b8 * 128
            mk = mrow_ref[b8:b8 + 1, :T2]
            for hh in range(H):
                q = qkv[r0:r0 + T2, hh * DK:(hh + 1) * DK]
                k = qkv[r0:r0 + T2, D + hh * DK:D + (hh + 1) * DK]
                vv = qkv[r0:r0 + T2, 2 * D + hh * DK:2 * D + (hh + 1) * DK]
                ph = pp[:T2, hh * DK:(hh + 1) * DK]
                uh = W['pos_u'][hh:hh + 1, :]
                vh = W['pos_v'][hh:hh + 1, :]
                s = (_bdot_t(q + uh, k) + _bdot_t(q + vh, ph)) * iscale
                s = jnp.where(mk > 0, s, -1e9)
                s = s - jnp.max(s, axis=-1, keepdims=True)
                es = jnp.exp(s)
                a = es / jnp.sum(es, axis=-1, keepdims=True)
                a = jnp.where(mk > 0, a, 0.0)
                scr[r0:r0 + T2, hh * DK:(hh + 1) * DK] = _bdot(a, vv)
        x2 = x1 + _bdot(scr[...], W['att_o_w'][...]) + W['att_o_b'][...]

        y = _ln_in(x2, W['ln_conv_g'][...], W['ln_conv_b'][...])
        y = y * mc
        t = _bdot(y, W['conv_pw1_w'][...]) + W['conv_pw1_b'][...]
        y = t[:, :D] * jax.nn.sigmoid(t[:, D:])
        z7 = jnp.zeros((pad, D), jnp.float32)
        for b8 in range(B):
            r0 = b8 * 128
            yp = jnp.concatenate([z7, y[r0:r0 + T2, :], z7], axis=0)
            acc = jnp.zeros((T2, D), jnp.float32)
            for kk in range(KER):
                acc = acc + yp[kk:kk + T2, :] * W['conv_dw'][kk:kk + 1, :]
            scr[r0:r0 + T2, :] = acc
        y = _ln_in(scr[...] + W['conv_dw_b'][...],
                   W['conv_ln_g'][...], W['conv_ln_b'][...])
        y = y * jax.nn.sigmoid(y)
        y = _bdot(y, W['conv_pw2_w'][...]) + W['conv_pw2_b'][...]
        x3 = x2 + y * mc

        y = _ln_in(x3, W['ln_ff_g'][...], W['ln_ff_b'][...])
        logits = jnp.dot(y, W['gate_w'][...], preferred_element_type=jnp.float32)
        logits = logits + W['gate_b'][...]
        mx = jnp.max(logits, -1, keepdims=True)
        ee = jnp.exp(logits - mx)
        probs = ee / jnp.sum(ee, -1, keepdims=True)
        gval = jnp.max(probs, -1, keepdims=True)
        iota_e = jax.lax.broadcasted_iota(jnp.int32, (NPAD, E), 1)
        cand = jnp.where(probs >= gval, iota_e, E)
        fidx = jnp.min(cand, -1, keepdims=True)
        row = jax.lax.broadcasted_iota(jnp.int32, (NPAD, 1), 0)
        validf = (jnp.bitwise_and(row, 127) < T2).astype(jnp.float32)
        onehot = (iota_e == fidx).astype(jnp.float32) * validf
        r2 = jax.lax.broadcasted_iota(jnp.int32, (NPAD, NPAD), 0)
        c2 = jax.lax.broadcasted_iota(jnp.int32, (NPAD, NPAD), 1)
        tri = (c2 <= r2).astype(jnp.float32)
        csum = jnp.dot(tri, onehot, preferred_element_type=jnp.float32)
        mypos = jnp.sum(csum * onehot, -1, keepdims=True) - 1.0
        keep = (mypos < CAP).astype(jnp.float32)
        kept = keep * validf * (mypos >= 0.0).astype(jnp.float32)
        dst = jnp.where(kept > 0, fidx.astype(jnp.float32) * CAPP + mypos, -1.0)
        slot_iota = jax.lax.broadcasted_iota(jnp.int32, (NPAD, E * CAPP), 1)
        Dt = (slot_iota.astype(jnp.float32) == dst).astype(jnp.float32)
        xslots = jax.lax.dot_general(
            Dt.astype(jnp.bfloat16), y.astype(jnp.bfloat16),
            (((0,), (0,)), ((), ())), preferred_element_type=jnp.float32)
        oparts = []
        for ei in range(E):
            he = _bdot(xslots[ei * CAPP:(ei + 1) * CAPP], W['exp_w1'][ei]) + W['exp_b1'][ei]
            he = he * jax.nn.sigmoid(he)
            oparts.append(_bdot(he, W['exp_w2'][ei]) + W['exp_b2'][ei])
        oslots = jnp.concatenate(oparts, axis=0)
        comb = _bdot(Dt, oslots)
        x4 = x3 + 0.5 * (comb * gval)
        x4 = _ln_in(x4, W['ln_final_g'][...], W['ln_final_b'][...])
        if extra:
            x4 = _ln_in(x4, W['after_g'][...], W['after_b'][...])
        o_ref[...] = x4

    vals = {
        'mac_w1': Lw['mac_w1'], 'mac_b1': Lw['mac_b1'].reshape(1, FF),
        'mac_w2': Lw['mac_w2'], 'mac_b2': Lw['mac_b2'].reshape(1, D),
        'qkv_w': jnp.concatenate([Lw['att_q_w'], Lw['att_k_w'], Lw['att_v_w']], axis=1),
        'qkv_b': jnp.concatenate([Lw['att_q_b'], Lw['att_k_b'], Lw['att_v_b']]).reshape(1, 3 * D),
        'att_pos_w': Lw['att_pos_w'], 'pos_u': Lw['pos_u'], 'pos_v': Lw['pos_v'],
        'att_o_w': Lw['att_o_w'], 'att_o_b': Lw['att_o_b'].reshape(1, D),
        'conv_pw1_w': Lw['conv_pw1_w'], 'conv_pw1_b': Lw['conv_pw1_b'].reshape(1, 2 * D),
        'conv_dw': jnp.transpose(Lw['conv_dw_w'].reshape(D, KER)),
        'conv_dw_b': Lw['conv_dw_b'].reshape(1, D),
        'conv_ln_g': Lw['conv_ln_g'].reshape(1, D), 'conv_ln_b': Lw['conv_ln_b'].reshape(1, D),
        'conv_pw2_w': Lw['conv_pw2_w'], 'conv_pw2_b': Lw['conv_pw2_b'].reshape(1, D),
        'gate_w': Lw['gate_w'], 'gate_b': Lw['gate_b'].reshape(1, E),
        'exp_w1': Lw['exp_w1'], 'exp_b1': Lw['exp_b1'].reshape(E, 1, EXP),
        'exp_w2': Lw['exp_w2'], 'exp_b2': Lw['exp_b2'].reshape(E, 1, D),
        'ln_ff_mac_g': Lw['ln_ff_mac_g'].reshape(1, D), 'ln_ff_mac_b': Lw['ln_ff_mac_b'].reshape(1, D),
        'ln_mha_g': Lw['ln_mha_g'].reshape(1, D), 'ln_mha_b': Lw['ln_mha_b'].reshape(1, D),
        'ln_conv_g': Lw['ln_conv_g'].reshape(1, D), 'ln_conv_b': Lw['ln_conv_b'].reshape(1, D),
        'ln_ff_g': Lw['ln_ff_g'].reshape(1, D), 'ln_ff_b': Lw['ln_ff_b'].reshape(1, D),
        'ln_final_g': Lw['ln_final_g'].reshape(1, D), 'ln_final_b': Lw['ln_final_b'].reshape(1, D),
    }
    if extra:
        vals['after_g'] = after[0].reshape(1, D)
        vals['after_b'] = after[1].reshape(1, D)
    ins = [x, mrow, mcol, pemb] + [vals[nm] for nm in names]
    return _pallas_call(
        body,
        out_shape=jax.ShapeDtypeStruct((NPAD, D), jnp.float32),
        scratch_shapes=[pltpu.VMEM((NPAD, D), jnp.float32)],
    )(*ins)


def _moe_route(xin, gg, gb, gw, gbias):
    def body(x_ref, gg_ref, gb_ref, gw_ref, gbias_ref, y_ref, dst_ref,
             src_ref, sc_ref):
        xb = x_ref[...]
        y = _ln_in(xb, gg_ref[...], gb_ref[...])
        y_ref[...] = y
        logits = jnp.dot(y, gw_ref[...], preferred_element_type=jnp.float32)
        logits = logits + gbias_ref[...]
        mx = jnp.max(logits, -1, keepdims=True)
        ee = jnp.exp(logits - mx)
        probs = ee / jnp.sum(ee, -1, keepdims=True)
        gval = jnp.max(probs, -1, keepdims=True)
        iota_e = jax.lax.broadcasted_iota(jnp.int32, (NPAD, E), 1)
        cand = jnp.where(probs >= gval, iota_e, E)
        fidx = jnp.min(cand, -1, keepdims=True)
        row = jax.lax.broadcasted_iota(jnp.int32, (NPAD, 1), 0)
        validf = (row < N_TOK).astype(jnp.float32)
        onehot = (iota_e == fidx).astype(jnp.float32) * validf
        r2 = jax.lax.broadcasted_iota(jnp.int32, (NPAD, NPAD), 0)
        c2 = jax.lax.broadcasted_iota(jnp.int32, (NPAD, NPAD), 1)
        tri = (c2 <= r2).astype(jnp.float32)
        csum = jnp.dot(tri, onehot, preferred_element_type=jnp.float32)
        mypos = jnp.sum(csum * onehot, -1, keepdims=True) - 1.0
        keep = (mypos < CAP).astype(jnp.float32)
        kept = keep * validf * (mypos >= 0.0).astype(jnp.float32)
        dstm = jnp.where(kept > 0, fidx.astype(jnp.float32) * CAPP + mypos, -1.0)
        slot_iota = jax.lax.broadcasted_iota(jnp.int32, (NPAD, E * CAPP), 1)
        Dt = (slot_iota.astype(jnp.float32) == dstm).astype(jnp.float32)
        tok = row.astype(jnp.float32)
        src = jax.lax.dot_general(Dt, tok, (((0,), (0,)), ((), ())),
                                  preferred_element_type=jnp.float32)
        dst_ref[...] = jnp.where(kept > 0, dstm, 0.0).astype(jnp.int32)
        src_ref[...] = src.astype(jnp.int32)
        sc_ref[...] = gval * kept

    return _pallas_call(
        body,
        out_shape=(
            jax.ShapeDtypeStruct((NPAD, D), jnp.float32),
            jax.ShapeDtypeStruct((NPAD, 1), jnp.int32),
            jax.ShapeDtypeStruct((E * CAPP, 1), jnp.int32),
            jax.ShapeDtypeStruct((NPAD, 1), jnp.float32),
        ),
    )(xin, gg, gb, gw, gbias)


def _sc_gather(table, idx, n_rows):
    info = plsc.get_sparse_core_info()
    nc, ns = info.num_cores, info.num_subcores
    nw = nc * ns
    bpw = n_rows // nw
    mesh = plsc.VectorSubcoreMesh(core_axis_name="c", subcore_axis_name="s")

    @functools.partial(
        pl.kernel, mesh=mesh,
        out_type=jax.ShapeDtypeStruct((n_rows, D), jnp.float32),
        scratch_types=[
            pltpu.VMEM((bpw,), jnp.int32),
            pltpu.VMEM((bpw, D), jnp.float32),
            pltpu.SemaphoreType.DMA,
        ],
    )
    def k(table_hbm, idx_hbm, out_hbm, idx_v, rows_v, sem):
        wid = jax.lax.axis_index("s") * nc + jax.lax.axis_index("c")
        base = wid * bpw
        pltpu.sync_copy(idx_hbm.at[pl.ds(base, bpw)], idx_v)
        pltpu.async_copy(table_hbm.at[idx_v], rows_v, sem).wait()
        pltpu.sync_copy(rows_v, out_hbm.at[pl.ds(base, bpw)])

    return k(table, idx)


def _expert_ffn(xslots, w1, b1, w2, b2):
    def body(x_ref, w1_ref, b1_ref, w2_ref, b2_ref, o_ref):
        h = _bdot(x_ref[...], w1_ref[0]) + b1_ref[0]
        h = h * jax.nn.sigmoid(h)
        o_ref[...] = _bdot(h, w2_ref[0]) + b2_ref[0]

    return _pallas_call(
        body,
        grid=(E,),
        in_specs=[
            pl.BlockSpec((CAPP, D), lambda e: (e, 0)),
            pl.BlockSpec((1, D, EXP), lambda e: (e, 0, 0)),
            pl.BlockSpec((1, 1, EXP), lambda e: (e, 0, 0)),
            pl.BlockSpec((1, EXP, D), lambda e: (e, 0, 0)),
            pl.BlockSpec((1, 1, D), lambda e: (e, 0, 0)),
        ],
        out_specs=pl.BlockSpec((CAPP, D), lambda e: (e, 0)),
        out_shape=jax.ShapeDtypeStruct((E * CAPP, D), jnp.float32),
    )(xslots, w1, b1, w2, b2)


def _moe_combine(xin, rows, scale, fg, fb, ag=None, ab=None):
    extra = ag is not None

    def body(*refs):
        x_ref, r_ref, s_ref, fg_ref, fb_ref = refs[:5]
        o_ref = refs[-1]
        x4 = x_ref[...] + 0.5 * (r_ref[...] * s_ref[...])
        x4 = _ln_in(x4, fg_ref[...], fb_ref[...])
        if extra:
            x4 = _ln_in(x4, refs[5][...], refs[6][...])
        o_ref[...] = x4

    ins = [xin, rows, scale, fg, fb]
    if extra:
        ins += [ag, ab]
    return _pallas_call(
        body,
        out_shape=jax.ShapeDtypeStruct((NPAD, D), jnp.float32),
    )(*ins)


def _moe_sc(xin, gg, gb, gw, gbias, w1, b1, w2, b2, fg, fb, ag=None, ab=None):
    y, dst, src, scale = _moe_route(xin, gg, gb, gw, gbias)
    xslots = _sc_gather(y, src.reshape(E * CAPP), E * CAPP)
    oslots = _expert_ffn(xslots, w1, b1, w2, b2)
    rows = _sc_gather(oslots, dst.reshape(NPAD), NPAD)
    return _moe_combine(xin, rows, scale, fg, fb, ag, ab)


def _rel_pos_enc_np(t):
    pos = np.arange(t)[:, None].astype(np.float32)
    div = np.exp(np.arange(0, D, 2).astype(np.float32) * -(np.log(10000.0) / D))
    pe = np.zeros((t, D), np.float32)
    pe[:, 0::2] = np.sin(pos * div)
    pe[:, 1::2] = np.cos(pos * div)
    return pe


def _subsample(xs, params):
    xsp = jnp.pad(xs, ((0, 0), (0, 4), (0, 4)))
    phase_stacks = []
    for p in range(2):
        for q in range(2):
            pats = []
            for kh in range(3):
                for kw in range(3):
                    sl = xsp[:, 2 * p + kh::4, :][:, :128, 2 * q + kw::4][:, :, :20]
                    pats.append(sl)
            phase_stacks.append(jnp.stack(pats, axis=-1))
    patches = jnp.concatenate(phase_stacks, axis=1)
    w1r = jnp.transpose(params['conv1_w'], (2, 3, 1, 0)).reshape(9, D)
    w2r = jnp.transpose(params['conv2_w'], (2, 3, 1, 0)).reshape(9, D, D)
    owr = params['out_w'].reshape(D, F2, D).transpose(1, 0, 2)
    b1 = params['conv1_b'].reshape(1, 1, D)
    b2 = params['conv2_b'].reshape(1, 1, D)
    ob = params['out_b'].reshape(1, D)
    scale = float(np.sqrt(D))

    def body(p_ref, w1_ref, b1_ref, w2_ref, b2_ref, ow_ref, ob_ref, o_ref):
        ph = _bdot(p_ref[0], w1_ref[...]) + b1_ref[...]
        ph = jnp.maximum(ph, 0.0)
        acc = jnp.zeros((T2, F2, D), jnp.float32)
        for kh in range(3):
            p = kh & 1
            to = kh >> 1
            for kw in range(3):
                q = kw & 1
                fo = kw >> 1
                base = (2 * p + q) * 128 + to
                sl = ph[base:base + T2, fo:fo + F2, :]
                acc = acc + _bdot(sl, w2_ref[kh * 3 + kw])
        z = jnp.maximum(acc + b2_ref[...], 0.0)
        out = ob_ref[...]
        for f in range(F2):
            out = out + _bdot(z[:, f, :], ow_ref[f])
        o_ref[0] = out * scale

    x0 = _pallas_call(
        body,
        grid=(B,),
        in_specs=[
            pl.BlockSpec((1, 512, 20, 9), lambda i: (i, 0, 0, 0)),
            pl.BlockSpec((9, D), lambda i: (0, 0)),
            pl.BlockSpec((1, 1, D), lambda i: (0, 0, 0)),
            pl.BlockSpec((9, D, D), lambda i: (0, 0, 0)),
            pl.BlockSpec((1, 1, D), lambda i: (0, 0, 0)),
            pl.BlockSpec((F2, D, D), lambda i: (0, 0, 0)),
            pl.BlockSpec((1, D), lambda i: (0, 0)),
        ],
        out_specs=pl.BlockSpec((1, T2, D), lambda i: (i, 0, 0)),
        out_shape=jax.ShapeDtypeStruct((B, T2, D), jnp.float32),
    )(patches, w1r, b1, w2r, b2, owr, ob)
    return x0.reshape(N_TOK, D)


def _layer(x0, pos_emb, mrow, mcol, L, after=None):
    qkv_w = jnp.concatenate([L['att_q_w'], L['att_k_w'], L['att_v_w']], axis=1)
    qkv_b = jnp.concatenate([L['att_q_b'], L['att_k_b'], L['att_v_b']], axis=0)

    h = _mm(x0, L['mac_w1'], L['mac_b1'], ln=(L['ln_ff_mac_g'], L['ln_ff_mac_b']),
            act='swish')
    x1 = _mm(h, L['mac_w2'], L['mac_b2'], residual=x0, ys=0.5)
    qkv = _mm(x1, qkv_w, qkv_b, ln=(L['ln_mha_g'], L['ln_mha_b']))
    p = _mm(pos_emb, L['att_pos_w'], bm=128)
    att = _attention(qkv.reshape(B, T2, 3 * D), p, L['pos_u'], L['pos_v'], mrow)
    x2 = _mm(att.reshape(N_TOK, D), L['att_o_w'], L['att_o_b'], residual=x1)
    x3 = _conv_module(x2.reshape(B, T2, D), mcol,
                      L['ln_conv_g'].reshape(1, D), L['ln_conv_b'].reshape(1, D),
                      L['conv_pw1_w'], L['conv_pw1_b'].reshape(1, 2 * D),
                      jnp.transpose(L['conv_dw_w'].reshape(D, KER)),
                      L['conv_dw_b'].reshape(1, D),
                      L['conv_ln_g'].reshape(1, D), L['conv_ln_b'].reshape(1, D),
                      L['conv_pw2_w'], L['conv_pw2_b'].reshape(1, D))
    x3f = jnp.pad(x3.reshape(N_TOK, D), ((0, NPAD - N_TOK), (0, 0)))
    kwargs = {}
    if after is not None:
        kwargs = dict(ag=after[0].reshape(1, D), ab=after[1].reshape(1, D))
    x4 = _moe(x3f,
              L['ln_ff_g'].reshape(1, D), L['ln_ff_b'].reshape(1, D),
              L['gate_w'], L['gate_b'].reshape(1, E),
              L['exp_w1'], L['exp_b1'].reshape(E, 1, EXP),
              L['exp_w2'], L['exp_b2'].reshape(E, 1, D),
              L['ln_final_g'].reshape(1, D), L['ln_final_b'].reshape(1, D),
              **kwargs)
    return x4[:N_TOK]


def _forward(xs, xs_lens, params):
    tm = jnp.arange(T_IN)[None, :] < xs_lens[:, None]
    masks = tm[:, 2::2][:, 2::2]
    mrow = jnp.pad(masks.astype(jnp.float32), ((0, 0), (0, 1)))
    mcol = mrow.reshape(NPAD, 1)
    pemb = jnp.pad(jnp.asarray(_rel_pos_enc_np(T2)), ((0, 1), (0, 0)))

    x = _subsample(xs, params)
    xp = jnp.pad(x.reshape(B, T2, D), ((0, 0), (0, 1), (0, 0))).reshape(NPAD, D)
    layers = params['layers']
    xp = _layer_fused(xp, mrow, mcol, pemb, layers[0])
    xp = _layer_fused(xp, mrow, mcol, pemb, layers[1],
                      after=(params['after_g'], params['after_b']))
    out = xp.reshape(B, 128, D)[:, :T2]
    return out, masks[:, None, :]


kernel = jax.jit(_forward)
